# baseline (speedup 1.0000x reference)
import functools

import jax
import jax.numpy as jnp
from jax import lax
from jax.experimental import pallas as pl
from jax.experimental.pallas import tpu as pltpu
from jax.experimental.pallas import tpu_sc as plsc

_D_MODEL = 768
_D_HALF = _D_MODEL // 2
_N_EXPERTS = 8
_D_FF = 1024
_D_SHARED = 1024
_AUX_COEFF = 0.01
_Z_COEFF = 0.001

_BM = 512
_BMS = 1024
_NTOK = 4096
_NBLK = _NTOK // _BM + _N_EXPERTS
_MPAD = _NBLK * _BM
_NBJ = 32


def _pack_rows(x):
    xb = x.astype(jnp.bfloat16)
    lo = lax.bitcast_convert_type(xb[:, :_D_HALF], jnp.uint16).astype(jnp.uint32)
    hi = lax.bitcast_convert_type(xb[:, _D_HALF:], jnp.uint16).astype(jnp.uint32)
    return lax.bitcast_convert_type(lo | (hi << 16), jnp.int32)


def _unpack_halves(p):
    pu = lax.bitcast_convert_type(p, jnp.uint32)
    lo = lax.bitcast_convert_type((pu & 0xFFFF).astype(jnp.uint16), jnp.bfloat16)
    hi = lax.bitcast_convert_type((pu >> 16).astype(jnp.uint16), jnp.bfloat16)
    return lo.astype(jnp.float32), hi.astype(jnp.float32)


def _unpack_rows(p):
    lo, hi = _unpack_halves(p)
    return jnp.concatenate([lo, hi], axis=1)


def _gating_body(xf_ref, wg_ref, xp_ref, dest_ref, bw_ref, bx_ref, bv_ref, loss_ref):
    n, e = _NTOK, _N_EXPERTS
    xf = xf_ref[...]
    wg = wg_ref[...]
    xbf = xf.astype(jnp.bfloat16)
    xp_ref[...] = _pack_rows(xbf)
    logits = lax.dot_general(
        xbf, wg.astype(jnp.bfloat16),
        (((1,), (0,)), ((), ())), preferred_element_type=jnp.float32)
    m = jnp.max(logits, axis=1, keepdims=True)
    ex = jnp.exp(logits - m)
    se = jnp.sum(ex, axis=1, keepdims=True)
    probs = ex / se
    lse = m + jnp.log(se)

    li = lax.broadcasted_iota(jnp.int32, (n, e), 1)
    eq = logits == m
    first = jnp.min(jnp.where(eq, li, e), axis=1, keepdims=True)
    oh = (li == first).astype(jnp.int32)

    ohf = oh.astype(jnp.float32)
    bs = 256
    tri_l = (lax.broadcasted_iota(jnp.int32, (bs, bs), 0)
             >= lax.broadcasted_iota(jnp.int32, (bs, bs), 1)).astype(jnp.float32)
    parts = []
    carry = jnp.zeros((1, e), jnp.float32)
    for b in range(n // bs):
        cs = lax.dot_general(tri_l, ohf[b * bs : (b + 1) * bs],
                             (((1,), (0,)), ((), ())),
                             preferred_element_type=jnp.float32)
        parts.append(cs + carry)
        carry = carry + cs[bs - 1 : bs]
    cf = jnp.concatenate(parts, axis=0)
    counts = cf[n - 1 : n, :].astype(jnp.int32)
    nb = (counts + (_BM - 1)) // _BM
    tri = (lax.broadcasted_iota(jnp.int32, (e, e), 0)
           < lax.broadcasted_iota(jnp.int32, (e, e), 1)).astype(jnp.float32)
    offs = lax.dot_general(
        (nb * _BM).astype(jnp.float32), tri, (((1,), (0,)), ((), ())),
        preferred_element_type=jnp.float32)
    startb = lax.dot_general(
        nb.astype(jnp.float32), tri, (((1,), (0,)), ((), ())),
        preferred_element_type=jnp.float32).astype(jnp.int32)
    total = jnp.sum(nb, axis=1, keepdims=True)

    destv = jnp.sum(
        jnp.where(oh == 1, offs + cf - 1.0, 0.0),
        axis=1, keepdims=True)
    dest_ref[...] = jnp.broadcast_to(destv.astype(jnp.int32), (n, e))

    ji = lax.broadcasted_iota(jnp.int32, (_NBJ, 1), 0)
    owner = jnp.sum((ji >= startb).astype(jnp.int32), axis=1, keepdims=True) - 1
    last_owner = jnp.sum(((total - 1) >= startb).astype(jnp.int32),
                         axis=1, keepdims=True) - 1
    valid = ji < total
    bw = jnp.where(valid, owner, last_owner)
    bx = jnp.where(valid, ji, total - 1)
    bw_ref[...] = jnp.broadcast_to(bw, (_NBJ, e))
    bx_ref[...] = jnp.broadcast_to(bx, (_NBJ, e))
    bv_ref[...] = jnp.broadcast_to(valid.astype(jnp.int32), (_NBJ, e))

    fe = counts.astype(jnp.float32) / float(n)
    pe = jnp.sum(probs, axis=0, keepdims=True) / float(n)
    aux = _AUX_COEFF * float(e) * jnp.sum(fe * pe, axis=1, keepdims=True)
    z = _Z_COEFF * jnp.sum(lse * lse, axis=0, keepdims=True) / float(n)
    loss_ref[...] = aux + z


def _ffn_body(bw_ref, bx_ref, bv_ref, xs_ref, w1_ref, w3_ref, w2_ref, y_ref):
    j = pl.program_id(0)

    @pl.when(bv_ref[j] == 1)
    def _():
        xb = _unpack_rows(xs_ref[...])
        h = jnp.dot(xb, w1_ref[0], preferred_element_type=jnp.float32)
        u = jnp.dot(xb, w3_ref[0], preferred_element_type=jnp.float32)
        g = (h / (1.0 + jnp.exp(-h))) * u
        y = jnp.dot(g, w2_ref[0], preferred_element_type=jnp.float32)
        y_ref[...] = _pack_rows(y)


def _shared_body(x_ref, g_ref, w1_ref, w3_ref, w2_ref, o_ref):
    xb = _unpack_rows(x_ref[...])
    h = jnp.dot(xb, w1_ref[...], preferred_element_type=jnp.float32)
    u = jnp.dot(xb, w3_ref[...], preferred_element_type=jnp.float32)
    s = (h / (1.0 + jnp.exp(-h))) * u
    o_ref[...] = (jnp.dot(s, w2_ref[...], preferred_element_type=jnp.float32)
                  + _unpack_rows(g_ref[...]))


def _sc_scatter(xp, dest):
    info = plsc.get_sparse_core_info()
    nc, ns = info.num_cores, info.num_subcores
    nw = nc * ns
    ch = _NTOK // nw
    mesh = plsc.VectorSubcoreMesh(core_axis_name="c", subcore_axis_name="s")

    @functools.partial(
        pl.kernel,
        out_type=jax.ShapeDtypeStruct((_MPAD, _D_HALF), jnp.int32),
        mesh=mesh,
        scratch_types=[
            pltpu.VMEM((ch,), jnp.int32),
            pltpu.VMEM((ch, _D_HALF), jnp.int32),
            pltpu.SemaphoreType.DMA,
        ],
    )
    def scat(x_hbm, dest_hbm, out_hbm, idx_v, rows_v, sem):
        wid = lax.axis_index("s") * nc + lax.axis_index("c")
        base = wid * ch
        pltpu.sync_copy(dest_hbm.at[pl.ds(base, ch)], idx_v)
        pltpu.sync_copy(x_hbm.at[pl.ds(base, ch)], rows_v)
        pltpu.async_copy(rows_v, out_hbm.at[idx_v], sem).wait()

    return scat(xp, dest)


def _sc_gather(yp, dest):
    info = plsc.get_sparse_core_info()
    nc, ns = info.num_cores, info.num_subcores
    nw = nc * ns
    ch = _NTOK // nw
    mesh = plsc.VectorSubcoreMesh(core_axis_name="c", subcore_axis_name="s")

    @functools.partial(
        pl.kernel,
        out_type=jax.ShapeDtypeStruct((_NTOK, _D_HALF), jnp.int32),
        mesh=mesh,
        scratch_types=[
            pltpu.VMEM((ch,), jnp.int32),
            pltpu.VMEM((ch, _D_HALF), jnp.int32),
            pltpu.SemaphoreType.DMA,
        ],
    )
    def gath(y_hbm, dest_hbm, out_hbm, idx_v, rows_v, sem):
        wid = lax.axis_index("s") * nc + lax.axis_index("c")
        base = wid * ch
        pltpu.sync_copy(dest_hbm.at[pl.ds(base, ch)], idx_v)
        pltpu.async_copy(y_hbm.at[idx_v], rows_v, sem).wait()
        pltpu.sync_copy(rows_v, out_hbm.at[pl.ds(base, ch)])

    return gath(yp, dest)


def kernel(x, Wg, W1, W3, W2, Ws1, Ws3, Ws2):
    bx_, tx_, d = x.shape
    xf = x.reshape(bx_ * tx_, d)

    xp, dest2d, bw2d, bx2d, bv2d, loss11 = pl.pallas_call(
        _gating_body,
        out_shape=[
            jax.ShapeDtypeStruct((_NTOK, _D_HALF), jnp.int32),
            jax.ShapeDtypeStruct((_NTOK, _N_EXPERTS), jnp.int32),
            jax.ShapeDtypeStruct((_NBJ, _N_EXPERTS), jnp.int32),
            jax.ShapeDtypeStruct((_NBJ, _N_EXPERTS), jnp.int32),
            jax.ShapeDtypeStruct((_NBJ, _N_EXPERTS), jnp.int32),
            jax.ShapeDtypeStruct((1, 1), jnp.float32),
        ],
    )(xf, Wg)

    dest = dest2d[:, 0]
    bw = bw2d[:_NBLK, 0]
    bxi = bx2d[:_NBLK, 0]
    bv = bv2d[:_NBLK, 0]

    sorted_xp = _sc_scatter(xp, dest)

    grid_spec = pltpu.PrefetchScalarGridSpec(
        num_scalar_prefetch=3,
        grid=(_NBLK,),
        in_specs=[
            pl.BlockSpec((_BM, _D_HALF), lambda j, bw, bx, bv: (bx[j], 0)),
            pl.BlockSpec((1, _D_MODEL, _D_FF), lambda j, bw, bx, bv: (bw[j], 0, 0)),
            pl.BlockSpec((1, _D_MODEL, _D_FF), lambda j, bw, bx, bv: (bw[j], 0, 0)),
            pl.BlockSpec((1, _D_FF, _D_MODEL), lambda j, bw, bx, bv: (bw[j], 0, 0)),
        ],
        out_specs=pl.BlockSpec((_BM, _D_HALF), lambda j, bw, bx, bv: (bx[j], 0)),
    )
    yp = pl.pallas_call(
        _ffn_body,
        grid_spec=grid_spec,
        out_shape=jax.ShapeDtypeStruct((_MPAD, _D_HALF), jnp.int32),
        compiler_params=pltpu.CompilerParams(
            dimension_semantics=("arbitrary",)),
    )(bw, bxi, bv, sorted_xp, W1, W3, W2)

    gp = _sc_gather(yp, dest)

    out = pl.pallas_call(
        _shared_body,
        grid=(_NTOK // _BMS,),
        in_specs=[
            pl.BlockSpec((_BMS, _D_HALF), lambda i: (i, 0)),
            pl.BlockSpec((_BMS, _D_HALF), lambda i: (i, 0)),
            pl.BlockSpec((_D_MODEL, _D_SHARED), lambda i: (0, 0)),
            pl.BlockSpec((_D_MODEL, _D_SHARED), lambda i: (0, 0)),
            pl.BlockSpec((_D_SHARED, _D_MODEL), lambda i: (0, 0)),
        ],
        out_specs=pl.BlockSpec((_BMS, _D_MODEL), lambda i: (i, 0)),
        out_shape=jax.ShapeDtypeStruct((_NTOK, _D_MODEL), jnp.float32),
        compiler_params=pltpu.CompilerParams(
            dimension_semantics=("arbitrary",)),
    )(xp, gp, Ws1, Ws3, Ws2)

    return out.reshape(bx_, tx_, d), loss11[0, 0]


# speedup vs baseline: 1.0116x; 1.0116x over previous
import functools

import jax
import jax.numpy as jnp
from jax import lax
from jax.experimental import pallas as pl
from jax.experimental.pallas import tpu as pltpu
from jax.experimental.pallas import tpu_sc as plsc

_D_MODEL = 768
_D_HALF = _D_MODEL // 2
_N_EXPERTS = 8
_D_FF = 1024
_D_SHARED = 1024
_AUX_COEFF = 0.01
_Z_COEFF = 0.001

_BM = 512
_BMS = 1024
_NTOK = 4096
_NBLK = _NTOK // _BM + _N_EXPERTS
_MPAD = _NBLK * _BM
_NBJ = 32


def _pack_rows(x):
    xb = x.astype(jnp.bfloat16)
    lo = lax.bitcast_convert_type(xb[:, :_D_HALF], jnp.uint16).astype(jnp.uint32)
    hi = lax.bitcast_convert_type(xb[:, _D_HALF:], jnp.uint16).astype(jnp.uint32)
    return lax.bitcast_convert_type(lo | (hi << 16), jnp.int32)


def _unpack_halves(p):
    pu = lax.bitcast_convert_type(p, jnp.uint32)
    lo = lax.bitcast_convert_type((pu & 0xFFFF).astype(jnp.uint16), jnp.bfloat16)
    hi = lax.bitcast_convert_type((pu >> 16).astype(jnp.uint16), jnp.bfloat16)
    return lo.astype(jnp.float32), hi.astype(jnp.float32)


def _unpack_rows(p):
    lo, hi = _unpack_halves(p)
    return jnp.concatenate([lo, hi], axis=1)


def _gating_body(xf_ref, wg_ref, xp_ref, dest_ref, bw_ref, bx_ref, bv_ref, loss_ref):
    n, e = _NTOK, _N_EXPERTS
    xf = xf_ref[...]
    wg = wg_ref[...]
    xbf = xf.astype(jnp.bfloat16)
    xp_ref[...] = _pack_rows(xbf)
    logits = lax.dot_general(
        xbf, wg.astype(jnp.bfloat16),
        (((1,), (0,)), ((), ())), preferred_element_type=jnp.float32)
    m = jnp.max(logits, axis=1, keepdims=True)
    ex = jnp.exp(logits - m)
    se = jnp.sum(ex, axis=1, keepdims=True)
    probs = ex / se
    lse = m + jnp.log(se)

    li = lax.broadcasted_iota(jnp.int32, (n, e), 1)
    eq = logits == m
    first = jnp.min(jnp.where(eq, li, e), axis=1, keepdims=True)
    oh = (li == first).astype(jnp.int32)

    c = oh
    k = 1
    while k < n:
        c = c + jnp.concatenate(
            [jnp.zeros((k, e), jnp.int32), c[: n - k]], axis=0)
        k *= 2
    cf = c.astype(jnp.float32)
    counts = c[n - 1 : n, :]
    nb = (counts + (_BM - 1)) // _BM
    tri = (lax.broadcasted_iota(jnp.int32, (e, e), 0)
           < lax.broadcasted_iota(jnp.int32, (e, e), 1)).astype(jnp.float32)
    offs = lax.dot_general(
        (nb * _BM).astype(jnp.float32), tri, (((1,), (0,)), ((), ())),
        preferred_element_type=jnp.float32)
    startb = lax.dot_general(
        nb.astype(jnp.float32), tri, (((1,), (0,)), ((), ())),
        preferred_element_type=jnp.float32).astype(jnp.int32)
    total = jnp.sum(nb, axis=1, keepdims=True)

    destv = jnp.sum(
        jnp.where(oh == 1, offs + cf - 1.0, 0.0),
        axis=1, keepdims=True)
    dest_ref[...] = jnp.broadcast_to(destv.astype(jnp.int32), (n, e))

    ji = lax.broadcasted_iota(jnp.int32, (_NBJ, 1), 0)
    owner = jnp.sum((ji >= startb).astype(jnp.int32), axis=1, keepdims=True) - 1
    last_owner = jnp.sum(((total - 1) >= startb).astype(jnp.int32),
                         axis=1, keepdims=True) - 1
    valid = ji < total
    bw = jnp.where(valid, owner, last_owner)
    bx = jnp.where(valid, ji, total - 1)
    bw_ref[...] = jnp.broadcast_to(bw, (_NBJ, e))
    bx_ref[...] = jnp.broadcast_to(bx, (_NBJ, e))
    bv_ref[...] = jnp.broadcast_to(valid.astype(jnp.int32), (_NBJ, e))

    fe = counts.astype(jnp.float32) / float(n)
    pe = jnp.sum(probs, axis=0, keepdims=True) / float(n)
    aux = _AUX_COEFF * float(e) * jnp.sum(fe * pe, axis=1, keepdims=True)
    z = _Z_COEFF * jnp.sum(lse * lse, axis=0, keepdims=True) / float(n)
    loss_ref[...] = aux + z


def _ffn_body(bw_ref, bx_ref, bv_ref, xs_ref, w1_ref, w3_ref, w2_ref, y_ref):
    j = pl.program_id(0)

    @pl.when(bv_ref[j] == 1)
    def _():
        xb = _unpack_rows(xs_ref[...])
        h = jnp.dot(xb, w1_ref[0], preferred_element_type=jnp.float32)
        u = jnp.dot(xb, w3_ref[0], preferred_element_type=jnp.float32)
        g = (h / (1.0 + jnp.exp(-h))) * u
        y = jnp.dot(g, w2_ref[0], preferred_element_type=jnp.float32)
        y_ref[...] = _pack_rows(y)


def _shared_body(x_ref, g_ref, w1_ref, w3_ref, w2_ref, o_ref):
    xb = _unpack_rows(x_ref[...])
    h = jnp.dot(xb, w1_ref[...], preferred_element_type=jnp.float32)
    u = jnp.dot(xb, w3_ref[...], preferred_element_type=jnp.float32)
    s = (h / (1.0 + jnp.exp(-h))) * u
    o_ref[...] = (jnp.dot(s, w2_ref[...], preferred_element_type=jnp.float32)
                  + _unpack_rows(g_ref[...]))


def _sc_scatter(xp, dest):
    info = plsc.get_sparse_core_info()
    nc, ns = info.num_cores, info.num_subcores
    nw = nc * ns
    ch = _NTOK // nw
    mesh = plsc.VectorSubcoreMesh(core_axis_name="c", subcore_axis_name="s")

    @functools.partial(
        pl.kernel,
        out_type=jax.ShapeDtypeStruct((_MPAD, _D_HALF), jnp.int32),
        mesh=mesh,
        scratch_types=[
            pltpu.VMEM((ch,), jnp.int32),
            pltpu.VMEM((ch, _D_HALF), jnp.int32),
            pltpu.SemaphoreType.DMA,
        ],
    )
    def scat(x_hbm, dest_hbm, out_hbm, idx_v, rows_v, sem):
        wid = lax.axis_index("s") * nc + lax.axis_index("c")
        base = wid * ch
        pltpu.sync_copy(dest_hbm.at[pl.ds(base, ch)], idx_v)
        pltpu.sync_copy(x_hbm.at[pl.ds(base, ch)], rows_v)
        pltpu.async_copy(rows_v, out_hbm.at[idx_v], sem).wait()

    return scat(xp, dest)


def _sc_gather(yp, dest):
    info = plsc.get_sparse_core_info()
    nc, ns = info.num_cores, info.num_subcores
    nw = nc * ns
    ch = _NTOK // nw
    mesh = plsc.VectorSubcoreMesh(core_axis_name="c", subcore_axis_name="s")

    @functools.partial(
        pl.kernel,
        out_type=jax.ShapeDtypeStruct((_NTOK, _D_HALF), jnp.int32),
        mesh=mesh,
        scratch_types=[
            pltpu.VMEM((ch,), jnp.int32),
            pltpu.VMEM((ch, _D_HALF), jnp.int32),
            pltpu.SemaphoreType.DMA,
        ],
    )
    def gath(y_hbm, dest_hbm, out_hbm, idx_v, rows_v, sem):
        wid = lax.axis_index("s") * nc + lax.axis_index("c")
        base = wid * ch
        pltpu.sync_copy(dest_hbm.at[pl.ds(base, ch)], idx_v)
        pltpu.async_copy(y_hbm.at[idx_v], rows_v, sem).wait()
        pltpu.sync_copy(rows_v, out_hbm.at[pl.ds(base, ch)])

    return gath(yp, dest)


def kernel(x, Wg, W1, W3, W2, Ws1, Ws3, Ws2):
    bx_, tx_, d = x.shape
    xf = x.reshape(bx_ * tx_, d)

    xp, dest2d, bw2d, bx2d, bv2d, loss11 = pl.pallas_call(
        _gating_body,
        out_shape=[
            jax.ShapeDtypeStruct((_NTOK, _D_HALF), jnp.int32),
            jax.ShapeDtypeStruct((_NTOK, _N_EXPERTS), jnp.int32),
            jax.ShapeDtypeStruct((_NBJ, _N_EXPERTS), jnp.int32),
            jax.ShapeDtypeStruct((_NBJ, _N_EXPERTS), jnp.int32),
            jax.ShapeDtypeStruct((_NBJ, _N_EXPERTS), jnp.int32),
            jax.ShapeDtypeStruct((1, 1), jnp.float32),
        ],
    )(xf, Wg)

    dest = dest2d[:, 0]
    bw = bw2d[:_NBLK, 0]
    bxi = bx2d[:_NBLK, 0]
    bv = bv2d[:_NBLK, 0]

    sorted_xp = _sc_scatter(xp, dest)

    grid_spec = pltpu.PrefetchScalarGridSpec(
        num_scalar_prefetch=3,
        grid=(_NBLK,),
        in_specs=[
            pl.BlockSpec((_BM, _D_HALF), lambda j, bw, bx, bv: (bx[j], 0)),
            pl.BlockSpec((1, _D_MODEL, _D_FF), lambda j, bw, bx, bv: (bw[j], 0, 0)),
            pl.BlockSpec((1, _D_MODEL, _D_FF), lambda j, bw, bx, bv: (bw[j], 0, 0)),
            pl.BlockSpec((1, _D_FF, _D_MODEL), lambda j, bw, bx, bv: (bw[j], 0, 0)),
        ],
        out_specs=pl.BlockSpec((_BM, _D_HALF), lambda j, bw, bx, bv: (bx[j], 0)),
    )
    yp = pl.pallas_call(
        _ffn_body,
        grid_spec=grid_spec,
        out_shape=jax.ShapeDtypeStruct((_MPAD, _D_HALF), jnp.int32),
        compiler_params=pltpu.CompilerParams(
            dimension_semantics=("arbitrary",)),
    )(bw, bxi, bv, sorted_xp, W1, W3, W2)

    gp = _sc_gather(yp, dest)

    out = pl.pallas_call(
        _shared_body,
        grid=(_NTOK // _BMS,),
        in_specs=[
            pl.BlockSpec((_BMS, _D_HALF), lambda i: (i, 0)),
            pl.BlockSpec((_BMS, _D_HALF), lambda i: (i, 0)),
            pl.BlockSpec((_D_MODEL, _D_SHARED), lambda i: (0, 0)),
            pl.BlockSpec((_D_MODEL, _D_SHARED), lambda i: (0, 0)),
            pl.BlockSpec((_D_SHARED, _D_MODEL), lambda i: (0, 0)),
        ],
        out_specs=pl.BlockSpec((_BMS, _D_MODEL), lambda i: (i, 0)),
        out_shape=jax.ShapeDtypeStruct((_NTOK, _D_MODEL), jnp.float32),
        compiler_params=pltpu.CompilerParams(
            dimension_semantics=("arbitrary",)),
    )(xp, gp, Ws1, Ws3, Ws2)

    return out.reshape(bx_, tx_, d), loss11[0, 0]
